# Initial kernel scaffold; baseline (speedup 1.0000x reference)
import jax
import jax.numpy as jnp
from jax.experimental import pallas as pl


def kernel(node_feats, edge_feats, edge_index, Wpn, bpn, Wpe1, bpe1, Wpe2, bpe2, Wet, bet, Wih1, bih1, Whh1, bhh1, Wpe_l, bpe_l, Wpn2, bpn2, Wih2, bih2, Whh2, bhh2, gamma, beta):
    raise NotImplementedError("write your pallas kernel here")


# speedup vs baseline: 4.4961x; 4.4961x over previous
import jax
import jax.numpy as jnp
from jax import lax
from jax.experimental import pallas as pl
from jax.experimental.pallas import tpu as pltpu
from jax.experimental.pallas import tpu_sc as plsc

N = 10000
E = 320000
NF = 128
EF = 16
GF = 200
DP = 256
SW = 64
NSL = DP // SW
NC = 2
NS = 16
NW = NC * NS
EPW = E // NW
KB = 400
NBLK = EPW // KB

_MESH = plsc.VectorSubcoreMesh(core_axis_name="c", subcore_axis_name="s")
_SC_PARAMS = pltpu.CompilerParams(needs_layout_passes=False,
                                  use_tc_tiling_on_sc=False)


def _loop(n, body):
    def f(i, c):
        body(i)
        return c
    lax.fori_loop(0, n, f, 0)


def _leaky(x):
    return jnp.maximum(x, 0.01 * x)


def _pad2(w, rows, cols):
    return jnp.zeros((rows, cols), jnp.float32).at[: w.shape[0], : w.shape[1]].set(w)


def _pad_row(b, cols):
    return jnp.zeros((1, cols), jnp.float32).at[0, : b.shape[0]].set(b)


def _sc_gather_body(table, srci, out, idx_v, rows_v, sem):
    wid = lax.axis_index("s") * NC + lax.axis_index("c")
    base = wid * EPW

    def blk(i):
        e0 = base + i * KB
        pltpu.sync_copy(srci.at[pl.ds(e0, KB)], idx_v)
        pltpu.async_copy(table.at[idx_v], rows_v, sem).wait()
        pltpu.sync_copy(rows_v, out.at[pl.ds(e0, KB)])

    _loop(NBLK, blk)


_sc_gather = pl.kernel(
    _sc_gather_body,
    compiler_params=_SC_PARAMS,
    out_type=jax.ShapeDtypeStruct((E, DP), jnp.float32),
    mesh=_MESH,
    scratch_types=[
        pltpu.VMEM((KB,), jnp.int32),
        pltpu.VMEM((KB, DP), jnp.float32),
        pltpu.SemaphoreType.DMA,
    ],
)


def _sc_seg_body(nscal, t_h, tabs_h, idxs_h, u_h, spart_h, tab_v, acc_v,
                 ii_v, tv_v, u_v):
    wid = lax.axis_index("s") * NC + lax.axis_index("c")
    base = wid * EPW

    for k in range(nscal):
        pltpu.sync_copy(tabs_h[k], tab_v.at[k])

    def z(i):
        acc_v[pl.ds(i * 16, 16)] = jnp.zeros((16,), jnp.float32)

    _loop(N // 16, z)

    def blk(i):
        e0 = base + i * KB
        for k in range(nscal):
            pltpu.sync_copy(idxs_h[k].at[pl.ds(e0, KB)], ii_v.at[k])
        if t_h is not None:
            pltpu.sync_copy(t_h.at[pl.ds(e0, KB)], tv_v)

        def grp(j):
            sl = pl.ds(j * 16, 16)
            idx0 = ii_v[0, sl]
            x = plsc.load_gather(tab_v.at[0], [idx0])
            for k in range(1, nscal):
                x = x + plsc.load_gather(tab_v.at[k], [ii_v[k, sl]])
            if t_h is not None:
                x = x + tv_v[sl]
            u = jnp.exp(jnp.maximum(x, 0.01 * x))
            u_v[sl] = u
            plsc.addupdate_scatter(acc_v, [idx0], u)

        _loop(KB // 16, grp)
        pltpu.sync_copy(u_v, u_h.at[pl.ds(e0, KB)])

    _loop(NBLK, blk)
    pltpu.sync_copy(acc_v, spart_h.at[wid])


def _make_sc_seg(nscal, with_t):
    def body(*refs):
        i = 0
        t_h = refs[i] if with_t else None
        i += 1 if with_t else 0
        tabs_h = refs[i:i + nscal]; i += nscal
        idxs_h = refs[i:i + nscal]; i += nscal
        u_h, spart_h, tab_v, acc_v, ii_v, tv_v, u_v = refs[i:]
        _sc_seg_body(nscal, t_h, tabs_h, idxs_h, u_h, spart_h, tab_v, acc_v,
                     ii_v, tv_v, u_v)

    return pl.kernel(
        body,
        compiler_params=_SC_PARAMS,
        out_type=(
            jax.ShapeDtypeStruct((E,), jnp.float32),
            jax.ShapeDtypeStruct((NW, N), jnp.float32),
        ),
        mesh=_MESH,
        scratch_types=[
            pltpu.VMEM((nscal, N), jnp.float32),
            pltpu.VMEM((N,), jnp.float32),
            pltpu.VMEM((nscal, KB), jnp.int32),
            pltpu.VMEM((KB,), jnp.float32),
            pltpu.VMEM((KB,), jnp.float32),
        ],
    )


_sc_seg1 = _make_sc_seg(1, True)
_sc_seg2 = _make_sc_seg(2, False)


def _edge_attn(u_v, r_v, di_v, a_v):
    def grp(j):
        sl = pl.ds(j * 16, 16)
        rg = plsc.load_gather(r_v, [di_v[sl]])
        a_v[sl] = u_v[sl] * rg

    _loop(KB // 16, grp)


def _scaled_scatter(rows_v, a_v, di_v, acc_sh):
    def sca(e):
        ab = plsc.load_gather(a_v, [jnp.full((16,), e, jnp.int32)])

        def col(j):
            sl = pl.ds(j * 16, 16)
            rows_v[e, sl] = rows_v[e, sl] * ab

        _loop(SW // 16, col)

    _loop(KB, sca)
    pltpu.sync_copy(rows_v, acc_sh.at[di_v], add=True)


def _make_sc_rowscat_lin(off):
    def body(rows_h, u_h, r_h, dsti_h, zero_h, out_h,
             r_v, di_v, u_v, a_v, rows_v, acc_sh, sem):
        c = lax.axis_index("c")
        s = lax.axis_index("s")
        wid = s * NC + c
        base = wid * EPW
        pltpu.sync_copy(r_h, r_v)

        @pl.when(s == 0)
        def _():
            pltpu.sync_copy(zero_h, acc_sh)

        plsc.subcore_barrier()

        def blk(i):
            e0 = base + i * KB
            pltpu.sync_copy(dsti_h.at[pl.ds(e0, KB)], di_v)
            pltpu.sync_copy(u_h.at[pl.ds(e0, KB)], u_v)
            pltpu.sync_copy(rows_h.at[pl.ds(e0, KB), pl.ds(off, SW)], rows_v)
            _edge_attn(u_v, r_v, di_v, a_v)
            _scaled_scatter(rows_v, a_v, di_v, acc_sh)

        _loop(NBLK, blk)
        plsc.subcore_barrier()

        @pl.when(s == 0)
        def _():
            pltpu.sync_copy(acc_sh, out_h.at[c])

    return pl.kernel(
        body,
        compiler_params=_SC_PARAMS,
        out_type=jax.ShapeDtypeStruct((NC, N, SW), jnp.float32),
        mesh=_MESH,
        scratch_types=[
            pltpu.VMEM((N,), jnp.float32),
            pltpu.VMEM((KB,), jnp.int32),
            pltpu.VMEM((KB,), jnp.float32),
            pltpu.VMEM((KB,), jnp.float32),
            pltpu.VMEM((KB, SW), jnp.float32),
            pltpu.VMEM_SHARED((N, SW), jnp.float32),
            pltpu.SemaphoreType.DMA,
        ],
    )


_sc_rowscat_lin = [_make_sc_rowscat_lin(k * SW) for k in range(NSL)]


def _sc_rowscat_gat_body(tab_h, u_h, r_h, srci_h, dsti_h, zero_h, out_h,
                         r_v, si_v, di_v, u_v, a_v, rows_v, acc_sh, sem):
    c = lax.axis_index("c")
    s = lax.axis_index("s")
    wid = s * NC + c
    base = wid * EPW
    pltpu.sync_copy(r_h, r_v)

    @pl.when(s == 0)
    def _():
        pltpu.sync_copy(zero_h, acc_sh)

    plsc.subcore_barrier()

    def blk(i):
        e0 = base + i * KB
        pltpu.sync_copy(srci_h.at[pl.ds(e0, KB)], si_v)
        pltpu.sync_copy(dsti_h.at[pl.ds(e0, KB)], di_v)
        pltpu.sync_copy(u_h.at[pl.ds(e0, KB)], u_v)
        pltpu.async_copy(tab_h.at[si_v], rows_v, sem).wait()
        _edge_attn(u_v, r_v, di_v, a_v)
        _scaled_scatter(rows_v, a_v, di_v, acc_sh)

    _loop(NBLK, blk)
    plsc.subcore_barrier()

    @pl.when(s == 0)
    def _():
        pltpu.sync_copy(acc_sh, out_h.at[c])


_sc_rowscat_gat = pl.kernel(
    _sc_rowscat_gat_body,
    compiler_params=_SC_PARAMS,
    out_type=jax.ShapeDtypeStruct((NC, N, SW), jnp.float32),
    mesh=_MESH,
    scratch_types=[
        pltpu.VMEM((N,), jnp.float32),
        pltpu.VMEM((KB,), jnp.int32),
        pltpu.VMEM((KB,), jnp.int32),
        pltpu.VMEM((KB,), jnp.float32),
        pltpu.VMEM((KB,), jnp.float32),
        pltpu.VMEM((KB, SW), jnp.float32),
        pltpu.VMEM_SHARED((N, SW), jnp.float32),
        pltpu.SemaphoreType.DMA,
    ],
)


def _rowdot(a, w):
    return jnp.sum(a * w[...].reshape(1, -1), axis=1, keepdims=True)


def _dot(a, b):
    return jnp.dot(a, b, preferred_element_type=jnp.float32,
                   precision=lax.Precision.HIGHEST)


def _tc_prep_nodes_body(nf, wpnT, bpn, a1T, wq, b2, hv_o, p_o, qd_o):
    x = nf[...]
    hv = _leaky(_dot(x, wpnT[...]) + bpn[...])
    hv_o[...] = hv
    p_o[...] = _dot(x, a1T[...])
    qd_o[...] = _rowdot(hv, wq) + b2[...]


def _tc_prep_edges_body(ef, b1T, bpe1, out):
    out[...] = _dot(ef[...], b1T[...]) + bpe1[...]


def _tc_logits_body(g, ep, w2, he1_o, t_o):
    he1 = _leaky(g[...] + ep[...])
    he1_o[...] = he1
    t_o[...] = _rowdot(he1, w2)


def _tc_recip_body(spart, r_o, s1_o):
    s = jnp.sum(spart[...], axis=0, keepdims=True)
    r = 1.0 / (s + 1e-16)
    r_o[...] = r
    s1_o[...] = s * r


def _elu(x):
    return jnp.where(x > 0, x, jnp.exp(x) - 1.0)


def _tc_mid_body(sp, s1, hv, wetT, betp,
                 wir, wiz, winn, bir, biz, binn,
                 whr, whz, whn, bhr, bhz, bhn,
                 wpd, bpd, wps, w2k0, w2k1, w2k2, w2k3, b2k,
                 h_o, pd_o, ps_o, hvp0_o, hvp1_o, hvp2_o, hvp3_o):
    spv = sp[...]
    wet = wetT[...]
    c = betp[...] * s1[...]
    for k in range(NSL):
        sk = spv[2 * k] + spv[2 * k + 1]
        c = c + _dot(sk, wet[k * SW:(k + 1) * SW, :])
    ctx = _elu(c)
    hvv = hv[...]
    r = jax.nn.sigmoid(_dot(ctx, wir[...]) + bir[...]
                       + _dot(hvv, whr[...]) + bhr[...])
    z = jax.nn.sigmoid(_dot(ctx, wiz[...]) + biz[...]
                       + _dot(hvv, whz[...]) + bhz[...])
    n = jnp.tanh(_dot(ctx, winn[...]) + binn[...]
                 + r * (_dot(hvv, whn[...]) + bhn[...]))
    h = jax.nn.relu((1.0 - z) * n + z * hvv)
    h_o[...] = h
    pd_o[...] = _rowdot(h, wpd) + bpd[...]
    ps_o[...] = _rowdot(h, wps)
    b2kv = b2k[...]
    hvp0_o[...] = _dot(h, w2k0[...]) + b2kv[:, 0 * SW:1 * SW]
    hvp1_o[...] = _dot(h, w2k1[...]) + b2kv[:, 1 * SW:2 * SW]
    hvp2_o[...] = _dot(h, w2k2[...]) + b2kv[:, 2 * SW:3 * SW]
    hvp3_o[...] = _dot(h, w2k3[...]) + b2kv[:, 3 * SW:4 * SW]


def _tc_gru2_body(c2, h, wih_r, wih_z, wih_n, bir, biz, binn,
                  whr, whz, whn, bhr, bhz, bhn, h2_o):
    c2v = c2[...]
    hv = h[...]
    wr = wih_r[...]
    wz = wih_z[...]
    wn = wih_n[...]
    gr = bir[...] + _dot(hv, whr[...]) + bhr[...]
    gz = biz[...] + _dot(hv, whz[...]) + bhz[...]
    gnh = _dot(hv, whn[...]) + bhn[...]
    gn = binn[...]
    for k in range(NSL):
        ek = _elu(c2v[2 * k] + c2v[2 * k + 1])
        gr = gr + _dot(ek, wr[k * SW:(k + 1) * SW, :])
        gz = gz + _dot(ek, wz[k * SW:(k + 1) * SW, :])
        gn = gn + _dot(ek, wn[k * SW:(k + 1) * SW, :])
    r = jax.nn.sigmoid(gr)
    z = jax.nn.sigmoid(gz)
    n = jnp.tanh(gn + r * gnh)
    h2_o[...] = jax.nn.relu((1.0 - z) * n + z * hv)


def _tc_norm_body(h2, h, gam, bet, out_o):
    h2v = h2[...]
    mu = jnp.mean(h2v, axis=0, keepdims=True)
    var = jnp.mean((h2v - mu) ** 2, axis=0, keepdims=True)
    h2n = (h2v - mu) / jnp.sqrt(var + 1e-5) * gam[...] + bet[...]
    out_o[...] = (h[...] + h2n)[:, :GF]


def _call_full(body, out_shapes):
    return pl.pallas_call(body, out_shape=out_shapes)


def kernel(node_feats, edge_feats, edge_index, Wpn, bpn, Wpe1, bpe1, Wpe2,
           bpe2, Wet, bet, Wih1, bih1, Whh1, bhh1, Wpe_l, bpe_l, Wpn2, bpn2,
           Wih2, bih2, Whh2, bhh2, gamma, beta):
    f32 = jnp.float32
    src = edge_index[0].astype(jnp.int32)
    dst = edge_index[1].astype(jnp.int32)

    wpnT = _pad2(Wpn.T, NF, DP)
    bpn_p = _pad_row(bpn, DP)
    a1T = _pad2(Wpe1[:, :NF].T, NF, DP)
    b1T = _pad2(Wpe1[:, NF:].T, EF, DP)
    bpe1_p = _pad_row(bpe1, DP)
    wq = _pad2(Wpe2[0, :GF][:, None], DP, 1)
    b2 = bpe2.reshape(1, 1).astype(f32)
    w2p = _pad2(Wpe2[0, GF:][:, None], DP, 1)
    wetT = _pad2(Wet.T, DP, DP)
    bet_p = _pad_row(bet, DP)

    def gru_weights(Wih, bih, Whh, bhh):
        ws = [_pad2(Wih[g * GF:(g + 1) * GF].T, DP, DP) for g in range(3)]
        bs = [_pad_row(bih[g * GF:(g + 1) * GF], DP) for g in range(3)]
        whs = [_pad2(Whh[g * GF:(g + 1) * GF].T, DP, DP) for g in range(3)]
        bhs = [_pad_row(bhh[g * GF:(g + 1) * GF], DP) for g in range(3)]
        return ws, bs, whs, bhs

    wi1, bi1, wh1, bh1 = gru_weights(Wih1, bih1, Whh1, bhh1)
    wi2, bi2, wh2, bh2 = gru_weights(Wih2, bih2, Whh2, bhh2)
    wpd = _pad2(Wpe_l[0, :GF][:, None], DP, 1)
    bpd = bpe_l.reshape(1, 1).astype(f32)
    wps = _pad2(Wpe_l[0, GF:][:, None], DP, 1)
    wpn2T = _pad2(Wpn2.T, DP, DP)
    bpn2_p = _pad_row(bpn2, DP)
    gam_p = _pad_row(gamma, DP)
    beta_p = _pad_row(beta, DP)
    zero_sl = jnp.zeros((N, SW), f32)

    BN1 = 2000

    def _row1(i):
        return (i, 0)

    def _rep1(i):
        return (0, 0)

    hv_new, P, qd2 = pl.pallas_call(
        _tc_prep_nodes_body,
        grid=(N // BN1,),
        in_specs=[pl.BlockSpec((BN1, NF), _row1),
                  pl.BlockSpec((NF, DP), _rep1),
                  pl.BlockSpec((1, DP), _rep1),
                  pl.BlockSpec((NF, DP), _rep1),
                  pl.BlockSpec((DP, 1), _rep1),
                  pl.BlockSpec((1, 1), _rep1)],
        out_specs=(pl.BlockSpec((BN1, DP), _row1),
                   pl.BlockSpec((BN1, DP), _row1),
                   pl.BlockSpec((BN1, 1), _row1)),
        out_shape=(jax.ShapeDtypeStruct((N, DP), f32),
                   jax.ShapeDtypeStruct((N, DP), f32),
                   jax.ShapeDtypeStruct((N, 1), f32)),
    )(node_feats, wpnT, bpn_p, a1T, wq, b2)

    BE = 4000
    epb = pl.pallas_call(
        _tc_prep_edges_body,
        grid=(E // BE,),
        in_specs=[pl.BlockSpec((BE, EF), lambda i: (i, 0)),
                  pl.BlockSpec((EF, DP), lambda i: (0, 0)),
                  pl.BlockSpec((1, DP), lambda i: (0, 0))],
        out_specs=pl.BlockSpec((BE, DP), lambda i: (i, 0)),
        out_shape=jax.ShapeDtypeStruct((E, DP), f32),
    )(edge_feats, b1T, bpe1_p)

    G = _sc_gather(P, src)

    BE2 = 2000
    he1, t2 = pl.pallas_call(
        _tc_logits_body,
        grid=(E // BE2,),
        in_specs=[pl.BlockSpec((BE2, DP), lambda i: (i, 0)),
                  pl.BlockSpec((BE2, DP), lambda i: (i, 0)),
                  pl.BlockSpec((DP, 1), lambda i: (0, 0))],
        out_specs=(pl.BlockSpec((BE2, DP), lambda i: (i, 0)),
                   pl.BlockSpec((BE2, 1), lambda i: (i, 0))),
        out_shape=(jax.ShapeDtypeStruct((E, DP), f32),
                   jax.ShapeDtypeStruct((E, 1), f32)),
    )(G, epb, w2p)

    u, spart = _sc_seg1(t2.reshape(E), qd2.reshape(N), dst)

    r1n, s11n = _call_full(
        _tc_recip_body,
        (jax.ShapeDtypeStruct((1, N), f32),
         jax.ShapeDtypeStruct((1, N), f32)),
    )(spart)
    r_vec = r1n.reshape(N)

    s_parts = [_sc_rowscat_lin[k](he1, u, r_vec, dst, zero_sl)
               for k in range(NSL)]
    sp = jnp.concatenate(s_parts, axis=0)

    BN = 1000

    def _row(i):
        return (i, 0)

    def _rep2(i):
        return (0, 0)

    def _sp3(i):
        return (0, i, 0)

    _w = lambda shape: pl.BlockSpec(shape, _rep2)
    h, pd2, ps2, hvp0, hvp1, hvp2, hvp3 = pl.pallas_call(
        _tc_mid_body,
        grid=(N // BN,),
        in_specs=[pl.BlockSpec((NSL * NC, BN, SW), _sp3),
                  pl.BlockSpec((BN, 1), _row),
                  pl.BlockSpec((BN, DP), _row),
                  _w((DP, DP)), _w((1, DP)),
                  _w((DP, DP)), _w((DP, DP)), _w((DP, DP)),
                  _w((1, DP)), _w((1, DP)), _w((1, DP)),
                  _w((DP, DP)), _w((DP, DP)), _w((DP, DP)),
                  _w((1, DP)), _w((1, DP)), _w((1, DP)),
                  _w((DP, 1)), _w((1, 1)), _w((DP, 1)),
                  _w((DP, SW)), _w((DP, SW)), _w((DP, SW)), _w((DP, SW)),
                  _w((1, DP))],
        out_specs=(pl.BlockSpec((BN, DP), _row),
                   pl.BlockSpec((BN, 1), _row),
                   pl.BlockSpec((BN, 1), _row),
                   pl.BlockSpec((BN, SW), _row),
                   pl.BlockSpec((BN, SW), _row),
                   pl.BlockSpec((BN, SW), _row),
                   pl.BlockSpec((BN, SW), _row)),
        out_shape=(jax.ShapeDtypeStruct((N, DP), f32),
                   jax.ShapeDtypeStruct((N, 1), f32),
                   jax.ShapeDtypeStruct((N, 1), f32),
                   jax.ShapeDtypeStruct((N, SW), f32),
                   jax.ShapeDtypeStruct((N, SW), f32),
                   jax.ShapeDtypeStruct((N, SW), f32),
                   jax.ShapeDtypeStruct((N, SW), f32)),
    )(sp, s11n.reshape(N, 1), hv_new, wetT, bet_p,
      wi1[0], wi1[1], wi1[2], bi1[0], bi1[1], bi1[2],
      wh1[0], wh1[1], wh1[2], bh1[0], bh1[1], bh1[2],
      wpd, bpd, wps,
      wpn2T[:, 0 * SW:1 * SW], wpn2T[:, 1 * SW:2 * SW],
      wpn2T[:, 2 * SW:3 * SW], wpn2T[:, 3 * SW:4 * SW], bpn2_p)

    u2, spart2 = _sc_seg2(pd2.reshape(N), ps2.reshape(N), dst, src)

    r21n, _unused = _call_full(
        _tc_recip_body,
        (jax.ShapeDtypeStruct((1, N), f32),
         jax.ShapeDtypeStruct((1, N), f32)),
    )(spart2)
    r2_vec = r21n.reshape(N)

    hvps = [hvp0, hvp1, hvp2, hvp3]
    c_parts = [_sc_rowscat_gat(hvps[k], u2, r2_vec, src, dst, zero_sl)
               for k in range(NSL)]
    c2 = jnp.concatenate(c_parts, axis=0)

    h2 = pl.pallas_call(
        _tc_gru2_body,
        grid=(N // BN,),
        in_specs=[pl.BlockSpec((NSL * NC, BN, SW), _sp3),
                  pl.BlockSpec((BN, DP), _row),
                  _w((DP, DP)), _w((DP, DP)), _w((DP, DP)),
                  _w((1, DP)), _w((1, DP)), _w((1, DP)),
                  _w((DP, DP)), _w((DP, DP)), _w((DP, DP)),
                  _w((1, DP)), _w((1, DP)), _w((1, DP))],
        out_specs=pl.BlockSpec((BN, DP), _row),
        out_shape=jax.ShapeDtypeStruct((N, DP), f32),
    )(c2, h,
      wi2[0], wi2[1], wi2[2], bi2[0], bi2[1], bi2[2],
      wh2[0], wh2[1], wh2[2], bh2[0], bh2[1], bh2[2])

    out = _call_full(
        _tc_norm_body,
        jax.ShapeDtypeStruct((N, GF), f32),
    )(h2, h, gam_p, beta_p)
    return out


# speedup vs baseline: 4.5201x; 1.0053x over previous
import jax
import jax.numpy as jnp
from jax import lax
from jax.experimental import pallas as pl
from jax.experimental.pallas import tpu as pltpu
from jax.experimental.pallas import tpu_sc as plsc

N = 10000
E = 320000
NF = 128
EF = 16
GF = 200
DP = 256
SW = 64
NSL = DP // SW
NC = 2
NS = 16
NW = NC * NS
EPW = E // NW
KB = 400
NBLK = EPW // KB

_MESH = plsc.VectorSubcoreMesh(core_axis_name="c", subcore_axis_name="s")
_SC_PARAMS = pltpu.CompilerParams(needs_layout_passes=False,
                                  use_tc_tiling_on_sc=False)


def _loop(n, body):
    def f(i, c):
        body(i)
        return c
    lax.fori_loop(0, n, f, 0)


def _leaky(x):
    return jnp.maximum(x, 0.01 * x)


def _pad2(w, rows, cols):
    return jnp.zeros((rows, cols), jnp.float32).at[: w.shape[0], : w.shape[1]].set(w)


def _pad_row(b, cols):
    return jnp.zeros((1, cols), jnp.float32).at[0, : b.shape[0]].set(b)


def _sc_gather_body(table, srci, out, idx_v, rows_v, sem):
    wid = lax.axis_index("s") * NC + lax.axis_index("c")
    base = wid * EPW

    def blk(i):
        e0 = base + i * KB
        pltpu.sync_copy(srci.at[pl.ds(e0, KB)], idx_v)
        pltpu.async_copy(table.at[idx_v], rows_v, sem).wait()
        pltpu.sync_copy(rows_v, out.at[pl.ds(e0, KB)])

    _loop(NBLK, blk)


_sc_gather = pl.kernel(
    _sc_gather_body,
    compiler_params=_SC_PARAMS,
    out_type=jax.ShapeDtypeStruct((E, DP), jnp.float32),
    mesh=_MESH,
    scratch_types=[
        pltpu.VMEM((KB,), jnp.int32),
        pltpu.VMEM((KB, DP), jnp.float32),
        pltpu.SemaphoreType.DMA,
    ],
)


def _sc_seg_body(nscal, t_h, tabs_h, idxs_h, u_h, spart_h, tab_v, acc_v,
                 ii_v, tv_v, u_v):
    wid = lax.axis_index("s") * NC + lax.axis_index("c")
    base = wid * EPW

    for k in range(nscal):
        pltpu.sync_copy(tabs_h[k], tab_v.at[k])

    def z(i):
        acc_v[pl.ds(i * 16, 16)] = jnp.zeros((16,), jnp.float32)

    _loop(N // 16, z)

    def blk(i):
        e0 = base + i * KB
        for k in range(nscal):
            pltpu.sync_copy(idxs_h[k].at[pl.ds(e0, KB)], ii_v.at[k])
        if t_h is not None:
            pltpu.sync_copy(t_h.at[pl.ds(e0, KB)], tv_v)

        def grp(j):
            sl = pl.ds(j * 16, 16)
            idx0 = ii_v[0, sl]
            x = plsc.load_gather(tab_v.at[0], [idx0])
            for k in range(1, nscal):
                x = x + plsc.load_gather(tab_v.at[k], [ii_v[k, sl]])
            if t_h is not None:
                x = x + tv_v[sl]
            u = jnp.exp(jnp.maximum(x, 0.01 * x))
            u_v[sl] = u
            plsc.addupdate_scatter(acc_v, [idx0], u)

        _loop(KB // 16, grp)
        pltpu.sync_copy(u_v, u_h.at[pl.ds(e0, KB)])

    _loop(NBLK, blk)
    pltpu.sync_copy(acc_v, spart_h.at[wid])


def _make_sc_seg(nscal, with_t):
    def body(*refs):
        i = 0
        t_h = refs[i] if with_t else None
        i += 1 if with_t else 0
        tabs_h = refs[i:i + nscal]; i += nscal
        idxs_h = refs[i:i + nscal]; i += nscal
        u_h, spart_h, tab_v, acc_v, ii_v, tv_v, u_v = refs[i:]
        _sc_seg_body(nscal, t_h, tabs_h, idxs_h, u_h, spart_h, tab_v, acc_v,
                     ii_v, tv_v, u_v)

    return pl.kernel(
        body,
        compiler_params=_SC_PARAMS,
        out_type=(
            jax.ShapeDtypeStruct((E,), jnp.float32),
            jax.ShapeDtypeStruct((NW, N), jnp.float32),
        ),
        mesh=_MESH,
        scratch_types=[
            pltpu.VMEM((nscal, N), jnp.float32),
            pltpu.VMEM((N,), jnp.float32),
            pltpu.VMEM((nscal, KB), jnp.int32),
            pltpu.VMEM((KB,), jnp.float32),
            pltpu.VMEM((KB,), jnp.float32),
        ],
    )


_sc_seg1 = _make_sc_seg(1, True)
_sc_seg2 = _make_sc_seg(2, False)


def _edge_attn(u_v, r_v, di_v, a_v):
    def grp(j):
        sl = pl.ds(j * 16, 16)
        rg = plsc.load_gather(r_v, [di_v[sl]])
        a_v[sl] = u_v[sl] * rg

    _loop(KB // 16, grp)


def _sc_attn_body(u_h, r_h, dsti_h, a_h, r_v, di_v, u_v, a_v):
    wid = lax.axis_index("s") * NC + lax.axis_index("c")
    base = wid * EPW
    pltpu.sync_copy(r_h, r_v)

    def blk(i):
        e0 = base + i * KB
        pltpu.sync_copy(dsti_h.at[pl.ds(e0, KB)], di_v)
        pltpu.sync_copy(u_h.at[pl.ds(e0, KB)], u_v)
        _edge_attn(u_v, r_v, di_v, a_v)
        pltpu.sync_copy(a_v, a_h.at[pl.ds(e0, KB)])

    _loop(NBLK, blk)


_sc_attn = pl.kernel(
    _sc_attn_body,
    compiler_params=_SC_PARAMS,
    out_type=jax.ShapeDtypeStruct((E,), jnp.float32),
    mesh=_MESH,
    scratch_types=[
        pltpu.VMEM((N,), jnp.float32),
        pltpu.VMEM((KB,), jnp.int32),
        pltpu.VMEM((KB,), jnp.float32),
        pltpu.VMEM((KB,), jnp.float32),
    ],
)


def _scaled_scatter(rows_v, a_v, di_v, acc_sh):
    def sca(e):
        ab = plsc.load_gather(a_v, [jnp.full((16,), e, jnp.int32)])

        def col(j):
            sl = pl.ds(j * 16, 16)
            rows_v[e, sl] = rows_v[e, sl] * ab

        _loop(SW // 16, col)

    _loop(KB, sca)
    pltpu.sync_copy(rows_v, acc_sh.at[di_v], add=True)


def _make_sc_rowscat_lin(off):
    def body(rows_h, dsti_h, zero_h, out_h, di_v, rows_v, acc_sh, sem):
        c = lax.axis_index("c")
        s = lax.axis_index("s")
        wid = s * NC + c
        base = wid * EPW

        @pl.when(s == 0)
        def _():
            pltpu.sync_copy(zero_h, acc_sh)

        plsc.subcore_barrier()

        def blk(i):
            e0 = base + i * KB
            pltpu.sync_copy(dsti_h.at[pl.ds(e0, KB)], di_v)
            pltpu.sync_copy(rows_h.at[pl.ds(e0, KB), pl.ds(off, SW)], rows_v)
            pltpu.sync_copy(rows_v, acc_sh.at[di_v], add=True)

        _loop(NBLK, blk)
        plsc.subcore_barrier()

        @pl.when(s == 0)
        def _():
            pltpu.sync_copy(acc_sh, out_h.at[c])

    return pl.kernel(
        body,
        compiler_params=_SC_PARAMS,
        out_type=jax.ShapeDtypeStruct((NC, N, SW), jnp.float32),
        mesh=_MESH,
        scratch_types=[
            pltpu.VMEM((KB,), jnp.int32),
            pltpu.VMEM((KB, SW), jnp.float32),
            pltpu.VMEM_SHARED((N, SW), jnp.float32),
            pltpu.SemaphoreType.DMA,
        ],
    )


_sc_rowscat_lin = [_make_sc_rowscat_lin(k * SW) for k in range(NSL)]


def _sc_rowscat_gat_body(tab_h, a_h, srci_h, dsti_h, zero_h, out_h,
                         si_v, di_v, a_v, rows_v, acc_sh, sem):
    c = lax.axis_index("c")
    s = lax.axis_index("s")
    wid = s * NC + c
    base = wid * EPW

    @pl.when(s == 0)
    def _():
        pltpu.sync_copy(zero_h, acc_sh)

    plsc.subcore_barrier()

    def blk(i):
        e0 = base + i * KB
        pltpu.sync_copy(srci_h.at[pl.ds(e0, KB)], si_v)
        pltpu.sync_copy(dsti_h.at[pl.ds(e0, KB)], di_v)
        pltpu.sync_copy(a_h.at[pl.ds(e0, KB)], a_v)
        pltpu.async_copy(tab_h.at[si_v], rows_v, sem).wait()
        _scaled_scatter(rows_v, a_v, di_v, acc_sh)

    _loop(NBLK, blk)
    plsc.subcore_barrier()

    @pl.when(s == 0)
    def _():
        pltpu.sync_copy(acc_sh, out_h.at[c])


_sc_rowscat_gat = pl.kernel(
    _sc_rowscat_gat_body,
    compiler_params=_SC_PARAMS,
    out_type=jax.ShapeDtypeStruct((NC, N, SW), jnp.float32),
    mesh=_MESH,
    scratch_types=[
        pltpu.VMEM((KB,), jnp.int32),
        pltpu.VMEM((KB,), jnp.int32),
        pltpu.VMEM((KB,), jnp.float32),
        pltpu.VMEM((KB, SW), jnp.float32),
        pltpu.VMEM_SHARED((N, SW), jnp.float32),
        pltpu.SemaphoreType.DMA,
    ],
)


def _rowdot(a, w):
    return jnp.sum(a * w[...].reshape(1, -1), axis=1, keepdims=True)


def _dot(a, b):
    return jnp.dot(a, b, preferred_element_type=jnp.float32,
                   precision=lax.Precision.HIGHEST)


def _tc_prep_nodes_body(nf, wpnT, bpn, a1T, wq, b2, hv_o, p_o, qd_o):
    x = nf[...]
    hv = _leaky(_dot(x, wpnT[...]) + bpn[...])
    hv_o[...] = hv
    p_o[...] = _dot(x, a1T[...])
    qd_o[...] = _rowdot(hv, wq) + b2[...]


def _tc_prep_edges_body(ef, b1T, bpe1, out):
    out[...] = _dot(ef[...], b1T[...]) + bpe1[...]


def _tc_scale_body(he1, a, out):
    out[...] = he1[...] * a[...]


def _tc_logits_body(g, ep, w2, he1_o, t_o):
    he1 = _leaky(g[...] + ep[...])
    he1_o[...] = he1
    t_o[...] = _rowdot(he1, w2)


def _tc_recip_body(spart, r_o, s1_o):
    s = jnp.sum(spart[...], axis=0, keepdims=True)
    r = 1.0 / (s + 1e-16)
    r_o[...] = r
    s1_o[...] = s * r


def _elu(x):
    return jnp.where(x > 0, x, jnp.exp(x) - 1.0)


def _tc_mid_body(sp0, sp1, sp2, sp3, s1, hv, wetT, betp,
                 wir, wiz, winn, bir, biz, binn,
                 whr, whz, whn, bhr, bhz, bhn,
                 wpd, bpd, wps, w2k0, w2k1, w2k2, w2k3, b2k,
                 h_o, pd_o, ps_o, hvp0_o, hvp1_o, hvp2_o, hvp3_o):
    wet = wetT[...]
    c = betp[...] * s1[...]
    for k, spk in enumerate((sp0, sp1, sp2, sp3)):
        v = spk[...]
        c = c + _dot(v[0] + v[1], wet[k * SW:(k + 1) * SW, :])
    ctx = _elu(c)
    hvv = hv[...]
    r = jax.nn.sigmoid(_dot(ctx, wir[...]) + bir[...]
                       + _dot(hvv, whr[...]) + bhr[...])
    z = jax.nn.sigmoid(_dot(ctx, wiz[...]) + biz[...]
                       + _dot(hvv, whz[...]) + bhz[...])
    n = jnp.tanh(_dot(ctx, winn[...]) + binn[...]
                 + r * (_dot(hvv, whn[...]) + bhn[...]))
    h = jax.nn.relu((1.0 - z) * n + z * hvv)
    h_o[...] = h
    pd_o[...] = _rowdot(h, wpd) + bpd[...]
    ps_o[...] = _rowdot(h, wps)
    b2kv = b2k[...]
    hvp0_o[...] = _dot(h, w2k0[...]) + b2kv[:, 0 * SW:1 * SW]
    hvp1_o[...] = _dot(h, w2k1[...]) + b2kv[:, 1 * SW:2 * SW]
    hvp2_o[...] = _dot(h, w2k2[...]) + b2kv[:, 2 * SW:3 * SW]
    hvp3_o[...] = _dot(h, w2k3[...]) + b2kv[:, 3 * SW:4 * SW]


def _tc_gru2_body(c20, c21, c22, c23, h, wih_r, wih_z, wih_n, bir, biz, binn,
                  whr, whz, whn, bhr, bhz, bhn, h2_o):
    hv = h[...]
    wr = wih_r[...]
    wz = wih_z[...]
    wn = wih_n[...]
    gr = bir[...] + _dot(hv, whr[...]) + bhr[...]
    gz = biz[...] + _dot(hv, whz[...]) + bhz[...]
    gnh = _dot(hv, whn[...]) + bhn[...]
    gn = binn[...]
    for k, ck in enumerate((c20, c21, c22, c23)):
        v = ck[...]
        ek = _elu(v[0] + v[1])
        gr = gr + _dot(ek, wr[k * SW:(k + 1) * SW, :])
        gz = gz + _dot(ek, wz[k * SW:(k + 1) * SW, :])
        gn = gn + _dot(ek, wn[k * SW:(k + 1) * SW, :])
    r = jax.nn.sigmoid(gr)
    z = jax.nn.sigmoid(gz)
    n = jnp.tanh(gn + r * gnh)
    h2_o[...] = jax.nn.relu((1.0 - z) * n + z * hv)


def _tc_norm_body(h2, h, gam, bet, out_o):
    h2v = h2[...]
    mu = jnp.mean(h2v, axis=0, keepdims=True)
    var = jnp.mean((h2v - mu) ** 2, axis=0, keepdims=True)
    h2n = (h2v - mu) / jnp.sqrt(var + 1e-5) * gam[...] + bet[...]
    out_o[...] = (h[...] + h2n)[:, :GF]


def _call_full(body, out_shapes):
    return pl.pallas_call(body, out_shape=out_shapes)


def kernel(node_feats, edge_feats, edge_index, Wpn, bpn, Wpe1, bpe1, Wpe2,
           bpe2, Wet, bet, Wih1, bih1, Whh1, bhh1, Wpe_l, bpe_l, Wpn2, bpn2,
           Wih2, bih2, Whh2, bhh2, gamma, beta):
    f32 = jnp.float32
    src = edge_index[0].astype(jnp.int32)
    dst = edge_index[1].astype(jnp.int32)

    wpnT = _pad2(Wpn.T, NF, DP)
    bpn_p = _pad_row(bpn, DP)
    a1T = _pad2(Wpe1[:, :NF].T, NF, DP)
    b1T = _pad2(Wpe1[:, NF:].T, EF, DP)
    bpe1_p = _pad_row(bpe1, DP)
    wq = _pad2(Wpe2[0, :GF][:, None], DP, 1)
    b2 = bpe2.reshape(1, 1).astype(f32)
    w2p = _pad2(Wpe2[0, GF:][:, None], DP, 1)
    wetT = _pad2(Wet.T, DP, DP)
    bet_p = _pad_row(bet, DP)

    def gru_weights(Wih, bih, Whh, bhh):
        ws = [_pad2(Wih[g * GF:(g + 1) * GF].T, DP, DP) for g in range(3)]
        bs = [_pad_row(bih[g * GF:(g + 1) * GF], DP) for g in range(3)]
        whs = [_pad2(Whh[g * GF:(g + 1) * GF].T, DP, DP) for g in range(3)]
        bhs = [_pad_row(bhh[g * GF:(g + 1) * GF], DP) for g in range(3)]
        return ws, bs, whs, bhs

    wi1, bi1, wh1, bh1 = gru_weights(Wih1, bih1, Whh1, bhh1)
    wi2, bi2, wh2, bh2 = gru_weights(Wih2, bih2, Whh2, bhh2)
    wpd = _pad2(Wpe_l[0, :GF][:, None], DP, 1)
    bpd = bpe_l.reshape(1, 1).astype(f32)
    wps = _pad2(Wpe_l[0, GF:][:, None], DP, 1)
    wpn2T = _pad2(Wpn2.T, DP, DP)
    bpn2_p = _pad_row(bpn2, DP)
    gam_p = _pad_row(gamma, DP)
    beta_p = _pad_row(beta, DP)
    zero_sl = jnp.zeros((N, SW), f32)

    BN1 = 2000

    def _row1(i):
        return (i, 0)

    def _rep1(i):
        return (0, 0)

    hv_new, P, qd2 = pl.pallas_call(
        _tc_prep_nodes_body,
        grid=(N // BN1,),
        in_specs=[pl.BlockSpec((BN1, NF), _row1),
                  pl.BlockSpec((NF, DP), _rep1),
                  pl.BlockSpec((1, DP), _rep1),
                  pl.BlockSpec((NF, DP), _rep1),
                  pl.BlockSpec((DP, 1), _rep1),
                  pl.BlockSpec((1, 1), _rep1)],
        out_specs=(pl.BlockSpec((BN1, DP), _row1),
                   pl.BlockSpec((BN1, DP), _row1),
                   pl.BlockSpec((BN1, 1), _row1)),
        out_shape=(jax.ShapeDtypeStruct((N, DP), f32),
                   jax.ShapeDtypeStruct((N, DP), f32),
                   jax.ShapeDtypeStruct((N, 1), f32)),
    )(node_feats, wpnT, bpn_p, a1T, wq, b2)

    BE = 4000
    epb = pl.pallas_call(
        _tc_prep_edges_body,
        grid=(E // BE,),
        in_specs=[pl.BlockSpec((BE, EF), lambda i: (i, 0)),
                  pl.BlockSpec((EF, DP), lambda i: (0, 0)),
                  pl.BlockSpec((1, DP), lambda i: (0, 0))],
        out_specs=pl.BlockSpec((BE, DP), lambda i: (i, 0)),
        out_shape=jax.ShapeDtypeStruct((E, DP), f32),
    )(edge_feats, b1T, bpe1_p)

    G = _sc_gather(P, src)

    BE2 = 2000
    he1, t2 = pl.pallas_call(
        _tc_logits_body,
        grid=(E // BE2,),
        in_specs=[pl.BlockSpec((BE2, DP), lambda i: (i, 0)),
                  pl.BlockSpec((BE2, DP), lambda i: (i, 0)),
                  pl.BlockSpec((DP, 1), lambda i: (0, 0))],
        out_specs=(pl.BlockSpec((BE2, DP), lambda i: (i, 0)),
                   pl.BlockSpec((BE2, 1), lambda i: (i, 0))),
        out_shape=(jax.ShapeDtypeStruct((E, DP), f32),
                   jax.ShapeDtypeStruct((E, 1), f32)),
    )(G, epb, w2p)

    u, spart = _sc_seg1(t2.reshape(E), qd2.reshape(N), dst)

    r1n, s11n = _call_full(
        _tc_recip_body,
        (jax.ShapeDtypeStruct((1, N), f32),
         jax.ShapeDtypeStruct((1, N), f32)),
    )(spart)
    r_vec = r1n.reshape(N)

    a1 = _sc_attn(u, r_vec, dst)
    BE2b = 2000
    he1a = pl.pallas_call(
        _tc_scale_body,
        grid=(E // BE2b,),
        in_specs=[pl.BlockSpec((BE2b, DP), lambda i: (i, 0)),
                  pl.BlockSpec((BE2b, 1), lambda i: (i, 0))],
        out_specs=pl.BlockSpec((BE2b, DP), lambda i: (i, 0)),
        out_shape=jax.ShapeDtypeStruct((E, DP), f32),
    )(he1, a1.reshape(E, 1))
    s_parts = [_sc_rowscat_lin[k](he1a, dst, zero_sl) for k in range(NSL)]

    BN = 1000

    def _row(i):
        return (i, 0)

    def _rep2(i):
        return (0, 0)

    def _sp3(i):
        return (0, i, 0)

    _w = lambda shape: pl.BlockSpec(shape, _rep2)
    h, pd2, ps2, hvp0, hvp1, hvp2, hvp3 = pl.pallas_call(
        _tc_mid_body,
        grid=(N // BN,),
        in_specs=[pl.BlockSpec((NC, BN, SW), _sp3),
                  pl.BlockSpec((NC, BN, SW), _sp3),
                  pl.BlockSpec((NC, BN, SW), _sp3),
                  pl.BlockSpec((NC, BN, SW), _sp3),
                  pl.BlockSpec((BN, 1), _row),
                  pl.BlockSpec((BN, DP), _row),
                  _w((DP, DP)), _w((1, DP)),
                  _w((DP, DP)), _w((DP, DP)), _w((DP, DP)),
                  _w((1, DP)), _w((1, DP)), _w((1, DP)),
                  _w((DP, DP)), _w((DP, DP)), _w((DP, DP)),
                  _w((1, DP)), _w((1, DP)), _w((1, DP)),
                  _w((DP, 1)), _w((1, 1)), _w((DP, 1)),
                  _w((DP, SW)), _w((DP, SW)), _w((DP, SW)), _w((DP, SW)),
                  _w((1, DP))],
        out_specs=(pl.BlockSpec((BN, DP), _row),
                   pl.BlockSpec((BN, 1), _row),
                   pl.BlockSpec((BN, 1), _row),
                   pl.BlockSpec((BN, SW), _row),
                   pl.BlockSpec((BN, SW), _row),
                   pl.BlockSpec((BN, SW), _row),
                   pl.BlockSpec((BN, SW), _row)),
        out_shape=(jax.ShapeDtypeStruct((N, DP), f32),
                   jax.ShapeDtypeStruct((N, 1), f32),
                   jax.ShapeDtypeStruct((N, 1), f32),
                   jax.ShapeDtypeStruct((N, SW), f32),
                   jax.ShapeDtypeStruct((N, SW), f32),
                   jax.ShapeDtypeStruct((N, SW), f32),
                   jax.ShapeDtypeStruct((N, SW), f32)),
    )(s_parts[0], s_parts[1], s_parts[2], s_parts[3],
      s11n.reshape(N, 1), hv_new, wetT, bet_p,
      wi1[0], wi1[1], wi1[2], bi1[0], bi1[1], bi1[2],
      wh1[0], wh1[1], wh1[2], bh1[0], bh1[1], bh1[2],
      wpd, bpd, wps,
      wpn2T[:, 0 * SW:1 * SW], wpn2T[:, 1 * SW:2 * SW],
      wpn2T[:, 2 * SW:3 * SW], wpn2T[:, 3 * SW:4 * SW], bpn2_p)

    u2, spart2 = _sc_seg2(pd2.reshape(N), ps2.reshape(N), dst, src)

    r21n, _unused = _call_full(
        _tc_recip_body,
        (jax.ShapeDtypeStruct((1, N), f32),
         jax.ShapeDtypeStruct((1, N), f32)),
    )(spart2)
    r2_vec = r21n.reshape(N)

    a2 = _sc_attn(u2, r2_vec, dst)
    hvps = [hvp0, hvp1, hvp2, hvp3]
    c_parts = [_sc_rowscat_gat(hvps[k], a2, src, dst, zero_sl)
               for k in range(NSL)]

    h2 = pl.pallas_call(
        _tc_gru2_body,
        grid=(N // BN,),
        in_specs=[pl.BlockSpec((NC, BN, SW), _sp3),
                  pl.BlockSpec((NC, BN, SW), _sp3),
                  pl.BlockSpec((NC, BN, SW), _sp3),
                  pl.BlockSpec((NC, BN, SW), _sp3),
                  pl.BlockSpec((BN, DP), _row),
                  _w((DP, DP)), _w((DP, DP)), _w((DP, DP)),
                  _w((1, DP)), _w((1, DP)), _w((1, DP)),
                  _w((DP, DP)), _w((DP, DP)), _w((DP, DP)),
                  _w((1, DP)), _w((1, DP)), _w((1, DP))],
        out_specs=pl.BlockSpec((BN, DP), _row),
        out_shape=jax.ShapeDtypeStruct((N, DP), f32),
    )(c_parts[0], c_parts[1], c_parts[2], c_parts[3], h,
      wi2[0], wi2[1], wi2[2], bi2[0], bi2[1], bi2[2],
      wh2[0], wh2[1], wh2[2], bh2[0], bh2[1], bh2[2])

    out = _call_full(
        _tc_norm_body,
        jax.ShapeDtypeStruct((N, GF), f32),
    )(h2, h, gam_p, beta_p)
    return out
